# baseline (speedup 1.0000x reference)
import functools
import math

import jax
import jax.numpy as jnp
import numpy as np
from jax import lax
from jax.experimental import pallas as pl
from jax.experimental.pallas import tpu as pltpu
from jax.experimental.pallas import tpu_sc as plsc

_D = 768
_E = 8
_NH = 12
_FF = 3072
_L = 2
_RH = 512
_PATH = 2
_MAXV = 1
_HD = _D // _NH


def _sc_embed(table, ids):
    S = ids.shape[0]
    D = table.shape[1]
    info = plsc.get_sparse_core_info()
    nw = info.num_cores * info.num_subcores
    b_per_w = S // nw
    mesh = plsc.VectorSubcoreMesh(core_axis_name="c", subcore_axis_name="s")

    @functools.partial(
        pl.kernel,
        out_type=jax.ShapeDtypeStruct((S, D), jnp.float32),
        mesh=mesh,
        scratch_types=[
            pltpu.VMEM((b_per_w,), jnp.int32),
            pltpu.VMEM((b_per_w, D), jnp.float32),
            pltpu.SemaphoreType.DMA,
        ],
    )
    def k(table_hbm, idx_hbm, out_hbm, idx_v, rows_v, sem):
        wid = lax.axis_index("s") * info.num_cores + lax.axis_index("c")
        base = wid * b_per_w
        pltpu.sync_copy(idx_hbm.at[pl.ds(base, b_per_w)], idx_v)
        pltpu.async_copy(table_hbm.at[idx_v], rows_v, sem).wait()
        pltpu.sync_copy(rows_v, out_hbm.at[pl.ds(base, b_per_w)])

    return k(table, ids)


def _bf(x):
    return x.astype(jnp.bfloat16) if x.dtype != jnp.bfloat16 else x


_NT = (((1,), (1,)), ((), ()))


def _mm(x, w, b, act=None, res=None, bn=256, out_dtype=jnp.float32):
    M, K = x.shape
    N = w.shape[0]

    def body(x_ref, w_ref, b_ref, *rest):
        if res is not None:
            r_ref, o_ref = rest
        else:
            (o_ref,) = rest
        acc = lax.dot_general(_bf(x_ref[...]), _bf(w_ref[...]), _NT,
                              preferred_element_type=jnp.float32)
        acc = acc + b_ref[...]
        if act == "relu":
            acc = jnp.maximum(acc, 0.0)
        elif act == "sigmoid":
            acc = jax.nn.sigmoid(acc)
        if res is not None:
            acc = acc + r_ref[...]
        o_ref[...] = acc.astype(out_dtype)

    in_specs = [
        pl.BlockSpec((M, K), lambda j: (0, 0)),
        pl.BlockSpec((bn, K), lambda j: (j, 0)),
        pl.BlockSpec((1, bn), lambda j: (0, j)),
    ]
    args = [x, w, b.reshape(1, N)]
    if res is not None:
        in_specs.append(pl.BlockSpec((M, bn), lambda j: (0, j)))
        args.append(res)
    return pl.pallas_call(
        body,
        grid=(N // bn,),
        in_specs=in_specs,
        out_specs=pl.BlockSpec((M, bn), lambda j: (0, j)),
        out_shape=jax.ShapeDtypeStruct((M, N), out_dtype),
    )(*args)


def _emm(x, W, l, e_arr, b, act=None, res=None, gate_split=None, bn=384,
         out_dtype=jnp.float32, dual=False):
    M, _ = x.shape
    N, K = W.shape[2], W.shape[3]

    def body(e_ref, x_ref, w_ref, b_ref, *rest):
        rest = list(rest)
        r_ref = rest.pop(0) if res is not None else None
        o_ref = rest.pop(0)
        o2_ref = rest.pop(0) if dual else None
        wb = _bf(w_ref[0, 0])
        if gate_split is not None:
            acc = lax.dot_general(_bf(x_ref[...]), wb[:, :gate_split], _NT,
                                  preferred_element_type=jnp.float32)
            acc = acc + lax.dot_general(_bf(r_ref[...]), wb[:, gate_split:],
                                        _NT, preferred_element_type=jnp.float32)
        else:
            acc = lax.dot_general(_bf(x_ref[...]), wb, _NT,
                                  preferred_element_type=jnp.float32)
        acc = acc + b_ref[...]
        if act == "relu":
            acc = jnp.maximum(acc, 0.0)
        elif act == "sigmoid":
            acc = jax.nn.sigmoid(acc)
        if res is not None and gate_split is None:
            acc = acc + r_ref[...]
        o_ref[...] = acc.astype(out_dtype)
        if dual:
            o2_ref[...] = acc.astype(jnp.bfloat16)

    in_specs = [
        pl.BlockSpec((M, x.shape[1]), lambda j, e: (0, 0)),
        pl.BlockSpec((1, 1, bn, K), lambda j, e: (e[0], l, j, 0)),
        pl.BlockSpec((1, bn), lambda j, e: (0, j)),
    ]
    args = [x, W, b.reshape(1, N)]
    if res is not None:
        if gate_split is not None:
            in_specs.append(
                pl.BlockSpec((M, K - gate_split), lambda j, e: (0, 0)))
        else:
            in_specs.append(pl.BlockSpec((M, bn), lambda j, e: (0, j)))
        args.append(res)
    out_shape = jax.ShapeDtypeStruct((M, N), out_dtype)
    out_specs = pl.BlockSpec((M, bn), lambda j, e: (0, j))
    if dual:
        out_shape = (out_shape, jax.ShapeDtypeStruct((M, N), jnp.bfloat16))
        out_specs = (out_specs, pl.BlockSpec((M, bn), lambda j, e: (0, j)))
    grid_spec = pltpu.PrefetchScalarGridSpec(
        num_scalar_prefetch=1,
        grid=(N // bn,),
        in_specs=in_specs,
        out_specs=out_specs,
    )
    return pl.pallas_call(
        body,
        grid_spec=grid_spec,
        out_shape=out_shape,
    )(e_arr, *args)


def _emm_ln(x, W, l, e_arr, b, res, lng, lnb, bm=512):
    M, K = x.shape
    N = W.shape[2]

    def body(e_ref, x_ref, w_ref, b_ref, r_ref, g_ref, bb_ref,
             o_ref, h_ref, hb_ref):
        acc = lax.dot_general(_bf(x_ref[...]), _bf(w_ref[0, 0]), _NT,
                              preferred_element_type=jnp.float32)
        xn = acc + b_ref[...] + r_ref[...]
        o_ref[...] = xn
        m = xn.mean(-1, keepdims=True)
        v = ((xn - m) ** 2).mean(-1, keepdims=True)
        h = (xn - m) / jnp.sqrt(v + 1e-5) * g_ref[...] + bb_ref[...]
        h_ref[...] = h
        hb_ref[...] = h.astype(jnp.bfloat16)

    blk = pl.BlockSpec((bm, N), lambda i, e: (i, 0))
    grid_spec = pltpu.PrefetchScalarGridSpec(
        num_scalar_prefetch=1,
        grid=(M // bm,),
        in_specs=[
            pl.BlockSpec((bm, K), lambda i, e: (i, 0)),
            pl.BlockSpec((1, 1, N, K), lambda i, e: (e[0], l, 0, 0)),
            pl.BlockSpec((1, N), lambda i, e: (0, 0)),
            blk,
            pl.BlockSpec((1, N), lambda i, e: (0, 0)),
            pl.BlockSpec((1, N), lambda i, e: (0, 0)),
        ],
        out_specs=(blk, blk, blk),
    )
    return pl.pallas_call(
        body,
        grid_spec=grid_spec,
        out_shape=(jax.ShapeDtypeStruct((M, N), jnp.float32),
                   jax.ShapeDtypeStruct((M, N), jnp.float32),
                   jax.ShapeDtypeStruct((M, N), jnp.bfloat16)),
    )(e_arr, x, W, b.reshape(1, N), res, lng.reshape(1, N), lnb.reshape(1, N))


def _ln(x, g, b, bm=512, dual=False, out_dtype=jnp.float32):
    M, D = x.shape

    def body(x_ref, g_ref, b_ref, *outs):
        xb = x_ref[...]
        m = xb.mean(-1, keepdims=True)
        v = ((xb - m) ** 2).mean(-1, keepdims=True)
        y = (xb - m) / jnp.sqrt(v + 1e-5) * g_ref[...] + b_ref[...]
        if dual:
            outs[0][...] = y
            outs[1][...] = y.astype(jnp.bfloat16)
        else:
            outs[0][...] = y.astype(out_dtype)

    out_shape = jax.ShapeDtypeStruct((M, D), out_dtype)
    out_specs = pl.BlockSpec((bm, D), lambda i: (i, 0))
    if dual:
        out_shape = (jax.ShapeDtypeStruct((M, D), jnp.float32),
                     jax.ShapeDtypeStruct((M, D), jnp.bfloat16))
        out_specs = (out_specs, pl.BlockSpec((bm, D), lambda i: (i, 0)))
    return pl.pallas_call(
        body,
        grid=(M // bm,),
        in_specs=[
            pl.BlockSpec((bm, D), lambda i: (i, 0)),
            pl.BlockSpec((1, D), lambda i: (0, 0)),
            pl.BlockSpec((1, D), lambda i: (0, 0)),
        ],
        out_specs=out_specs,
        out_shape=out_shape,
    )(x, g.reshape(1, D), b.reshape(1, D))


def _add(a, b, bm=512):
    M, D = a.shape

    def body(a_ref, b_ref, o_ref):
        o_ref[...] = a_ref[...] + b_ref[...]

    return pl.pallas_call(
        body,
        grid=(M // bm,),
        in_specs=[pl.BlockSpec((bm, D), lambda i: (i, 0))] * 2,
        out_specs=pl.BlockSpec((bm, D), lambda i: (i, 0)),
        out_shape=jax.ShapeDtypeStruct((M, D), jnp.float32),
    )(a, b)


def _gate_combine(x, h, gt, a, gg, gb, n2g, n2b, bm=512):
    M, D = x.shape

    def body(x_ref, h_ref, gt_ref, a_ref, g_ref, b_ref, g2_ref, b2_ref,
             o_ref, o2_ref):
        u = h_ref[...] + gt_ref[...] * a_ref[...]
        m = u.mean(-1, keepdims=True)
        v = ((u - m) ** 2).mean(-1, keepdims=True)
        xn = x_ref[...] + (u - m) / jnp.sqrt(v + 1e-5) * g_ref[...] + b_ref[...]
        o_ref[...] = xn
        m2 = xn.mean(-1, keepdims=True)
        v2 = ((xn - m2) ** 2).mean(-1, keepdims=True)
        h3 = (xn - m2) / jnp.sqrt(v2 + 1e-5) * g2_ref[...] + b2_ref[...]
        o2_ref[...] = h3.astype(jnp.bfloat16)

    vec = pl.BlockSpec((1, D), lambda i: (0, 0))
    blk = pl.BlockSpec((bm, D), lambda i: (i, 0))
    return pl.pallas_call(
        body,
        grid=(M // bm,),
        in_specs=[blk] * 4 + [vec] * 4,
        out_specs=(blk, blk),
        out_shape=(jax.ShapeDtypeStruct((M, D), jnp.float32),
                   jax.ShapeDtypeStruct((M, D), jnp.bfloat16)),
    )(x, h, gt, a, gg.reshape(1, D), gb.reshape(1, D),
      n2g.reshape(1, D), n2b.reshape(1, D))


def _attn_qkv(qkv):
    S = qkv.shape[0]
    bq = 256
    scale = 1.0 / math.sqrt(_HD)
    hp = _NH // 2

    ones = jnp.ones((S, 128), jnp.bfloat16)

    def body(q_ref, k_ref, v_ref, ones_ref, o_ref):
        q = q_ref[...] * jnp.bfloat16(scale)
        k = k_ref[...]
        v = v_ref[...]

        def one(qh, kh, vh):
            att = lax.dot_general(qh, kh, _NT,
                                  preferred_element_type=jnp.float32)
            pb = _bf(jnp.exp(att))
            s = jnp.dot(pb, ones_ref[...], preferred_element_type=jnp.float32)
            r = 1.0 / s[:, :1]
            return jnp.dot(pb, vh, preferred_element_type=jnp.float32) * r

        o1 = one(q[:, :_HD], k[:, :_HD], v[:, :_HD])
        o2 = one(q[:, _HD:], k[:, _HD:], v[:, _HD:])
        o_ref[...] = jnp.concatenate([o1, o2], axis=1).astype(jnp.bfloat16)

    return pl.pallas_call(
        body,
        grid=(hp, S // bq),
        in_specs=[
            pl.BlockSpec((bq, 2 * _HD), lambda h, i: (i, h)),
            pl.BlockSpec((S, 2 * _HD), lambda h, i: (0, hp + h)),
            pl.BlockSpec((S, 2 * _HD), lambda h, i: (0, 2 * hp + h)),
            pl.BlockSpec((S, 128), lambda h, i: (0, 0)),
        ],
        out_specs=pl.BlockSpec((bq, 2 * _HD), lambda h, i: (i, h)),
        out_shape=jax.ShapeDtypeStruct((S, _D), jnp.bfloat16),
    )(qkv, qkv, qkv, ones)


def _router(x, w1, b1, w2, b2, gain, bias):
    S, D = x.shape
    RH = w1.shape[0]
    EP = w2.shape[0]

    def body(x_ref, w1_ref, b1_ref, w2_ref, b2_ref, g_ref, bias_ref, o_ref):
        s = jnp.mean(x_ref[...], axis=0, keepdims=True)
        h = lax.dot_general(s, w1_ref[...], _NT,
                            preferred_element_type=jnp.float32) + b1_ref[...]
        h = jnp.maximum(h, 0.0)
        lg = lax.dot_general(h, w2_ref[...], _NT,
                             preferred_element_type=jnp.float32) + b2_ref[...]
        lg = jnp.clip(lg, -10.0, 10.0)
        lane = lax.broadcasted_iota(jnp.int32, (1, EP), 1)
        valid = lane < (_E + 1)
        cnt = float(_E + 1)
        m = jnp.sum(jnp.where(valid, lg, 0.0)) / cnt
        var = jnp.sum(jnp.where(valid, (lg - m) ** 2, 0.0)) / cnt
        lgn = (lg - m) / jnp.sqrt(var + 1e-5)
        score = lgn * g_ref[...] + bias_ref[...]
        top = jnp.max(score, axis=1, keepdims=True)
        cand = jnp.where(score >= top, lane, EP)
        o_ref[...] = jnp.min(cand, axis=1, keepdims=True)

    return pl.pallas_call(
        body,
        grid=(1,),
        in_specs=[
            pl.BlockSpec((S, D), lambda i: (0, 0)),
            pl.BlockSpec((RH, D), lambda i: (0, 0)),
            pl.BlockSpec((1, RH), lambda i: (0, 0)),
            pl.BlockSpec((EP, RH), lambda i: (0, 0)),
            pl.BlockSpec((1, EP), lambda i: (0, 0)),
            pl.BlockSpec((1, EP), lambda i: (0, 0)),
            pl.BlockSpec((1, EP), lambda i: (0, 0)),
        ],
        out_specs=pl.BlockSpec((1, 1), lambda i: (0, 0)),
        out_shape=jax.ShapeDtypeStruct((1, 1), jnp.int32),
    )(x, w1, b1.reshape(1, RH), w2, b2.reshape(1, EP), gain, bias)


def _pe_table(seq, d):
    pos = np.arange(seq)[:, None].astype(np.float32)
    div = np.exp(np.arange(0, d, 2).astype(np.float32) * (-math.log(10000.0) / d))
    pe = np.zeros((seq, d), np.float32)
    pe[:, 0::2] = np.sin(pos * div)
    pe[:, 1::2] = np.cos(pos * div)
    return jnp.asarray(pe)


def _expert(p, e, x):
    e_arr = e.reshape(1).astype(jnp.int32)

    def sl(name):
        return lax.dynamic_index_in_dim(p[name], e, 0, keepdims=False)

    bqkv, bo = sl("attn_bqkv"), sl("attn_bo")
    gb = sl("gate_b")
    gag, gab = sl("ga_g"), sl("ga_b")
    n1g, n1b = sl("norm1_g"), sl("norm1_b")
    n2g, n2b = sl("norm2_g"), sl("norm2_b")
    b1, b2 = sl("ffn_b1"), sl("ffn_b2")
    tag = sl("tag")

    h, hb = _ln(x, n1g[0], n1b[0], dual=True)
    for l in range(_L):
        qkv = _emm(hb, p["attn_Wqkv"], l, e_arr, bqkv[l],
                   out_dtype=jnp.bfloat16)
        ao = _attn_qkv(qkv)
        a, ab = _emm(ao, p["attn_Wo"], l, e_arr, bo[l], dual=True)
        gt = _emm(hb, p["gate_W"], l, e_arr, gb[l], act="sigmoid",
                  res=ab, gate_split=_D)
        x, h3 = _gate_combine(x, h, gt, a, gag[l], gab[l], n2g[l], n2b[l])
        f1 = _emm(h3, p["ffn_W1"], l, e_arr, b1[l], act="relu",
                  out_dtype=jnp.bfloat16)
        if l == _L - 1:
            x = _emm(f1, p["ffn_W2"], l, e_arr, b2[l] + tag, res=x)
        else:
            x, h, hb = _emm_ln(f1, p["ffn_W2"], l, e_arr, b2[l], x,
                               n1g[l + 1], n1b[l + 1])
    return x


def kernel(params, input_ids):
    p = params
    Bz, S = input_ids.shape
    ids = input_ids.reshape(S)

    emb = _sc_embed(p["embedding"], ids)
    x = _add(emb, _pe_table(S, _D))

    EP = 16
    w2p = jnp.zeros((EP, _RH), jnp.float32).at[: _E + 1].set(p["fc2_W"])
    b2p = jnp.zeros((EP,), jnp.float32).at[: _E + 1].set(p["fc2_b"])
    qv = jnp.zeros((EP,), jnp.float32).at[: _E + 1].set(p["q_values"])
    gpad = jnp.ones((EP,), jnp.float32).at[: _E + 1].set(p["rnorm_g"])
    bpad = jnp.zeros((EP,), jnp.float32).at[: _E + 1].set(p["rnorm_b"])
    lane_kill = jnp.where(jnp.arange(EP) < _E + 1, 0.0, -jnp.inf)

    visit = jnp.zeros((_E,), jnp.float32)
    for _ in range(_PATH):
        ext = jnp.concatenate([visit >= _MAXV, jnp.zeros((1,), bool)])
        mpad = jnp.zeros((EP,), jnp.float32).at[: _E + 1].set(
            jnp.where(ext, -jnp.inf, 0.0))
        bias = (bpad + mpad + qv + lane_kill).reshape(1, EP)
        act = _router(x, p["fc1_W"], p["fc1_b"], w2p, b2p,
                      gpad.reshape(1, EP), bias)
        action = act[0, 0]
        e = jnp.minimum(action, _E - 1)
        x = lax.cond(action < _E, lambda xx: _expert(p, e, xx),
                     lambda xx: xx, x)
        visit = visit + jax.nn.one_hot(action, _E + 1)[:_E]

    xb = _ln(x, p["fnorm_g"], p["fnorm_b"], out_dtype=jnp.bfloat16)
    logits = _mm(xb, p["lm_W"], p["lm_b"], bn=640)
    return logits.reshape(Bz, S, -1)


# speedup vs baseline: 1.1077x; 1.1077x over previous
import functools
import math

import jax
import jax.numpy as jnp
import numpy as np
from jax import lax
from jax.experimental import pallas as pl
from jax.experimental.pallas import tpu as pltpu
from jax.experimental.pallas import tpu_sc as plsc

_D = 768
_E = 8
_NH = 12
_FF = 3072
_L = 2
_RH = 512
_PATH = 2
_MAXV = 1
_HD = _D // _NH


def _sc_embed(table, ids):
    S = ids.shape[0]
    D = table.shape[1]
    info = plsc.get_sparse_core_info()
    nw = info.num_cores * info.num_subcores
    b_per_w = S // nw
    mesh = plsc.VectorSubcoreMesh(core_axis_name="c", subcore_axis_name="s")

    @functools.partial(
        pl.kernel,
        out_type=jax.ShapeDtypeStruct((S, D), jnp.float32),
        mesh=mesh,
        scratch_types=[
            pltpu.VMEM((b_per_w,), jnp.int32),
            pltpu.VMEM((b_per_w, D), jnp.float32),
            pltpu.SemaphoreType.DMA,
        ],
    )
    def k(table_hbm, idx_hbm, out_hbm, idx_v, rows_v, sem):
        wid = lax.axis_index("s") * info.num_cores + lax.axis_index("c")
        base = wid * b_per_w
        pltpu.sync_copy(idx_hbm.at[pl.ds(base, b_per_w)], idx_v)
        pltpu.async_copy(table_hbm.at[idx_v], rows_v, sem).wait()
        pltpu.sync_copy(rows_v, out_hbm.at[pl.ds(base, b_per_w)])

    return k(table, ids)


def _bf(x):
    return x.astype(jnp.bfloat16) if x.dtype != jnp.bfloat16 else x


_NT = (((1,), (1,)), ((), ()))


def _mm(x, w, b, act=None, res=None, bn=256, out_dtype=jnp.float32):
    M, K = x.shape
    N = w.shape[0]

    def body(x_ref, w_ref, b_ref, *rest):
        if res is not None:
            r_ref, o_ref = rest
        else:
            (o_ref,) = rest
        acc = lax.dot_general(_bf(x_ref[...]), _bf(w_ref[...]), _NT,
                              preferred_element_type=jnp.float32)
        acc = acc + b_ref[...]
        if act == "relu":
            acc = jnp.maximum(acc, 0.0)
        elif act == "sigmoid":
            acc = jax.nn.sigmoid(acc)
        if res is not None:
            acc = acc + r_ref[...]
        o_ref[...] = acc.astype(out_dtype)

    in_specs = [
        pl.BlockSpec((M, K), lambda j: (0, 0)),
        pl.BlockSpec((bn, K), lambda j: (j, 0)),
        pl.BlockSpec((1, bn), lambda j: (0, j)),
    ]
    args = [x, w, b.reshape(1, N)]
    if res is not None:
        in_specs.append(pl.BlockSpec((M, bn), lambda j: (0, j)))
        args.append(res)
    return pl.pallas_call(
        body,
        grid=(N // bn,),
        in_specs=in_specs,
        out_specs=pl.BlockSpec((M, bn), lambda j: (0, j)),
        out_shape=jax.ShapeDtypeStruct((M, N), out_dtype),
    )(*args)


def _emm(x, W, l, e_arr, b, act=None, res=None, gate_split=None, bn=384,
         out_dtype=jnp.float32, dual=False):
    M, _ = x.shape
    N, K = W.shape[2], W.shape[3]

    def body(e_ref, x_ref, w_ref, b_ref, *rest):
        rest = list(rest)
        r_ref = rest.pop(0) if res is not None else None
        o_ref = rest.pop(0)
        o2_ref = rest.pop(0) if dual else None
        wb = _bf(w_ref[0, 0])
        if gate_split is not None:
            acc = lax.dot_general(_bf(x_ref[...]), wb[:, :gate_split], _NT,
                                  preferred_element_type=jnp.float32)
            acc = acc + lax.dot_general(_bf(r_ref[...]), wb[:, gate_split:],
                                        _NT, preferred_element_type=jnp.float32)
        else:
            acc = lax.dot_general(_bf(x_ref[...]), wb, _NT,
                                  preferred_element_type=jnp.float32)
        acc = acc + b_ref[...]
        if act == "relu":
            acc = jnp.maximum(acc, 0.0)
        elif act == "sigmoid":
            acc = jax.nn.sigmoid(acc)
        if res is not None and gate_split is None:
            acc = acc + r_ref[...]
        o_ref[...] = acc.astype(out_dtype)
        if dual:
            o2_ref[...] = acc.astype(jnp.bfloat16)

    in_specs = [
        pl.BlockSpec((M, x.shape[1]), lambda j, e: (0, 0)),
        pl.BlockSpec((1, 1, bn, K), lambda j, e: (e[0], l, j, 0)),
        pl.BlockSpec((1, bn), lambda j, e: (0, j)),
    ]
    args = [x, W, b.reshape(1, N)]
    if res is not None:
        if gate_split is not None:
            in_specs.append(
                pl.BlockSpec((M, K - gate_split), lambda j, e: (0, 0)))
        else:
            in_specs.append(pl.BlockSpec((M, bn), lambda j, e: (0, j)))
        args.append(res)
    out_shape = jax.ShapeDtypeStruct((M, N), out_dtype)
    out_specs = pl.BlockSpec((M, bn), lambda j, e: (0, j))
    if dual:
        out_shape = (out_shape, jax.ShapeDtypeStruct((M, N), jnp.bfloat16))
        out_specs = (out_specs, pl.BlockSpec((M, bn), lambda j, e: (0, j)))
    grid_spec = pltpu.PrefetchScalarGridSpec(
        num_scalar_prefetch=1,
        grid=(N // bn,),
        in_specs=in_specs,
        out_specs=out_specs,
    )
    return pl.pallas_call(
        body,
        grid_spec=grid_spec,
        out_shape=out_shape,
    )(e_arr, *args)


def _emm_ln(x, W, l, e_arr, b, res, lng, lnb, bm=512):
    M, K = x.shape
    N = W.shape[2]

    def body(e_ref, x_ref, w_ref, b_ref, r_ref, g_ref, bb_ref,
             o_ref, h_ref, hb_ref):
        acc = lax.dot_general(_bf(x_ref[...]), _bf(w_ref[0, 0]), _NT,
                              preferred_element_type=jnp.float32)
        xn = acc + b_ref[...] + r_ref[...]
        o_ref[...] = xn
        m = xn.mean(-1, keepdims=True)
        v = ((xn - m) ** 2).mean(-1, keepdims=True)
        h = (xn - m) / jnp.sqrt(v + 1e-5) * g_ref[...] + bb_ref[...]
        h_ref[...] = h
        hb_ref[...] = h.astype(jnp.bfloat16)

    blk = pl.BlockSpec((bm, N), lambda i, e: (i, 0))
    grid_spec = pltpu.PrefetchScalarGridSpec(
        num_scalar_prefetch=1,
        grid=(M // bm,),
        in_specs=[
            pl.BlockSpec((bm, K), lambda i, e: (i, 0)),
            pl.BlockSpec((1, 1, N, K), lambda i, e: (e[0], l, 0, 0)),
            pl.BlockSpec((1, N), lambda i, e: (0, 0)),
            blk,
            pl.BlockSpec((1, N), lambda i, e: (0, 0)),
            pl.BlockSpec((1, N), lambda i, e: (0, 0)),
        ],
        out_specs=(blk, blk, blk),
    )
    return pl.pallas_call(
        body,
        grid_spec=grid_spec,
        out_shape=(jax.ShapeDtypeStruct((M, N), jnp.float32),
                   jax.ShapeDtypeStruct((M, N), jnp.float32),
                   jax.ShapeDtypeStruct((M, N), jnp.bfloat16)),
    )(e_arr, x, W, b.reshape(1, N), res, lng.reshape(1, N), lnb.reshape(1, N))


def _ln(x, g, b, bm=512, dual=False, out_dtype=jnp.float32):
    M, D = x.shape

    def body(x_ref, g_ref, b_ref, *outs):
        xb = x_ref[...]
        m = xb.mean(-1, keepdims=True)
        v = ((xb - m) ** 2).mean(-1, keepdims=True)
        y = (xb - m) / jnp.sqrt(v + 1e-5) * g_ref[...] + b_ref[...]
        if dual:
            outs[0][...] = y
            outs[1][...] = y.astype(jnp.bfloat16)
        else:
            outs[0][...] = y.astype(out_dtype)

    out_shape = jax.ShapeDtypeStruct((M, D), out_dtype)
    out_specs = pl.BlockSpec((bm, D), lambda i: (i, 0))
    if dual:
        out_shape = (jax.ShapeDtypeStruct((M, D), jnp.float32),
                     jax.ShapeDtypeStruct((M, D), jnp.bfloat16))
        out_specs = (out_specs, pl.BlockSpec((bm, D), lambda i: (i, 0)))
    return pl.pallas_call(
        body,
        grid=(M // bm,),
        in_specs=[
            pl.BlockSpec((bm, D), lambda i: (i, 0)),
            pl.BlockSpec((1, D), lambda i: (0, 0)),
            pl.BlockSpec((1, D), lambda i: (0, 0)),
        ],
        out_specs=out_specs,
        out_shape=out_shape,
    )(x, g.reshape(1, D), b.reshape(1, D))


def _add(a, b, bm=512):
    M, D = a.shape

    def body(a_ref, b_ref, o_ref):
        o_ref[...] = a_ref[...] + b_ref[...]

    return pl.pallas_call(
        body,
        grid=(M // bm,),
        in_specs=[pl.BlockSpec((bm, D), lambda i: (i, 0))] * 2,
        out_specs=pl.BlockSpec((bm, D), lambda i: (i, 0)),
        out_shape=jax.ShapeDtypeStruct((M, D), jnp.float32),
    )(a, b)


def _gate_combine(x, h, gt, a, gg, gb, n2g, n2b, bm=512):
    M, D = x.shape

    def body(x_ref, h_ref, gt_ref, a_ref, g_ref, b_ref, g2_ref, b2_ref,
             o_ref, o2_ref):
        u = h_ref[...] + gt_ref[...] * a_ref[...]
        m = u.mean(-1, keepdims=True)
        v = ((u - m) ** 2).mean(-1, keepdims=True)
        xn = x_ref[...] + (u - m) / jnp.sqrt(v + 1e-5) * g_ref[...] + b_ref[...]
        o_ref[...] = xn
        m2 = xn.mean(-1, keepdims=True)
        v2 = ((xn - m2) ** 2).mean(-1, keepdims=True)
        h3 = (xn - m2) / jnp.sqrt(v2 + 1e-5) * g2_ref[...] + b2_ref[...]
        o2_ref[...] = h3.astype(jnp.bfloat16)

    vec = pl.BlockSpec((1, D), lambda i: (0, 0))
    blk = pl.BlockSpec((bm, D), lambda i: (i, 0))
    return pl.pallas_call(
        body,
        grid=(M // bm,),
        in_specs=[blk] * 4 + [vec] * 4,
        out_specs=(blk, blk),
        out_shape=(jax.ShapeDtypeStruct((M, D), jnp.float32),
                   jax.ShapeDtypeStruct((M, D), jnp.bfloat16)),
    )(x, h, gt, a, gg.reshape(1, D), gb.reshape(1, D),
      n2g.reshape(1, D), n2b.reshape(1, D))


def _attn_qkv(qkv):
    S = qkv.shape[0]
    bq = 256
    scale = 1.0 / math.sqrt(_HD)
    hp = _NH // 2

    ones = jnp.ones((S, 128), jnp.bfloat16)

    def body(q_ref, k_ref, v_ref, ones_ref, o_ref):
        q = q_ref[...] * jnp.bfloat16(scale)
        k = k_ref[...]
        v = v_ref[...]

        def one(qh, kh, vh):
            att = lax.dot_general(qh, kh, _NT,
                                  preferred_element_type=jnp.float32)
            p = jnp.exp(att)
            r = 1.0 / p.sum(-1, keepdims=True)
            return jnp.dot(_bf(p), vh, preferred_element_type=jnp.float32) * r

        o1 = one(q[:, :_HD], k[:, :_HD], v[:, :_HD])
        o2 = one(q[:, _HD:], k[:, _HD:], v[:, _HD:])
        o_ref[...] = jnp.concatenate([o1, o2], axis=1).astype(jnp.bfloat16)

    return pl.pallas_call(
        body,
        grid=(hp, S // bq),
        in_specs=[
            pl.BlockSpec((bq, 2 * _HD), lambda h, i: (i, h)),
            pl.BlockSpec((S, 2 * _HD), lambda h, i: (0, hp + h)),
            pl.BlockSpec((S, 2 * _HD), lambda h, i: (0, 2 * hp + h)),
            pl.BlockSpec((S, 128), lambda h, i: (0, 0)),
        ],
        out_specs=pl.BlockSpec((bq, 2 * _HD), lambda h, i: (i, h)),
        out_shape=jax.ShapeDtypeStruct((S, _D), jnp.bfloat16),
    )(qkv, qkv, qkv, ones)


def _router(x, w1, b1, w2, b2, gain, bias):
    S, D = x.shape
    RH = w1.shape[0]
    EP = w2.shape[0]

    def body(x_ref, w1_ref, b1_ref, w2_ref, b2_ref, g_ref, bias_ref, o_ref):
        s = jnp.mean(x_ref[...], axis=0, keepdims=True)
        h = lax.dot_general(s, w1_ref[...], _NT,
                            preferred_element_type=jnp.float32) + b1_ref[...]
        h = jnp.maximum(h, 0.0)
        lg = lax.dot_general(h, w2_ref[...], _NT,
                             preferred_element_type=jnp.float32) + b2_ref[...]
        lg = jnp.clip(lg, -10.0, 10.0)
        lane = lax.broadcasted_iota(jnp.int32, (1, EP), 1)
        valid = lane < (_E + 1)
        cnt = float(_E + 1)
        m = jnp.sum(jnp.where(valid, lg, 0.0)) / cnt
        var = jnp.sum(jnp.where(valid, (lg - m) ** 2, 0.0)) / cnt
        lgn = (lg - m) / jnp.sqrt(var + 1e-5)
        score = lgn * g_ref[...] + bias_ref[...]
        top = jnp.max(score, axis=1, keepdims=True)
        cand = jnp.where(score >= top, lane, EP)
        o_ref[...] = jnp.min(cand, axis=1, keepdims=True)

    return pl.pallas_call(
        body,
        grid=(1,),
        in_specs=[
            pl.BlockSpec((S, D), lambda i: (0, 0)),
            pl.BlockSpec((RH, D), lambda i: (0, 0)),
            pl.BlockSpec((1, RH), lambda i: (0, 0)),
            pl.BlockSpec((EP, RH), lambda i: (0, 0)),
            pl.BlockSpec((1, EP), lambda i: (0, 0)),
            pl.BlockSpec((1, EP), lambda i: (0, 0)),
            pl.BlockSpec((1, EP), lambda i: (0, 0)),
        ],
        out_specs=pl.BlockSpec((1, 1), lambda i: (0, 0)),
        out_shape=jax.ShapeDtypeStruct((1, 1), jnp.int32),
    )(x, w1, b1.reshape(1, RH), w2, b2.reshape(1, EP), gain, bias)


def _pe_table(seq, d):
    pos = np.arange(seq)[:, None].astype(np.float32)
    div = np.exp(np.arange(0, d, 2).astype(np.float32) * (-math.log(10000.0) / d))
    pe = np.zeros((seq, d), np.float32)
    pe[:, 0::2] = np.sin(pos * div)
    pe[:, 1::2] = np.cos(pos * div)
    return jnp.asarray(pe)


def _expert(p, e, x):
    e_arr = e.reshape(1).astype(jnp.int32)

    def sl(name):
        return lax.dynamic_index_in_dim(p[name], e, 0, keepdims=False)

    bqkv, bo = sl("attn_bqkv"), sl("attn_bo")
    gb = sl("gate_b")
    gag, gab = sl("ga_g"), sl("ga_b")
    n1g, n1b = sl("norm1_g"), sl("norm1_b")
    n2g, n2b = sl("norm2_g"), sl("norm2_b")
    b1, b2 = sl("ffn_b1"), sl("ffn_b2")
    tag = sl("tag")

    h, hb = _ln(x, n1g[0], n1b[0], dual=True)
    for l in range(_L):
        qkv = _emm(hb, p["attn_Wqkv"], l, e_arr, bqkv[l],
                   out_dtype=jnp.bfloat16)
        ao = _attn_qkv(qkv)
        a, ab = _emm(ao, p["attn_Wo"], l, e_arr, bo[l], dual=True)
        gt = _emm(hb, p["gate_W"], l, e_arr, gb[l], act="sigmoid",
                  res=ab, gate_split=_D)
        x, h3 = _gate_combine(x, h, gt, a, gag[l], gab[l], n2g[l], n2b[l])
        f1 = _emm(h3, p["ffn_W1"], l, e_arr, b1[l], act="relu",
                  out_dtype=jnp.bfloat16)
        if l == _L - 1:
            x = _emm(f1, p["ffn_W2"], l, e_arr, b2[l] + tag, res=x)
        else:
            x, h, hb = _emm_ln(f1, p["ffn_W2"], l, e_arr, b2[l], x,
                               n1g[l + 1], n1b[l + 1])
    return x


def kernel(params, input_ids):
    p = params
    Bz, S = input_ids.shape
    ids = input_ids.reshape(S)

    emb = _sc_embed(p["embedding"], ids)
    x = _add(emb, _pe_table(S, _D))

    EP = 16
    w2p = jnp.zeros((EP, _RH), jnp.float32).at[: _E + 1].set(p["fc2_W"])
    b2p = jnp.zeros((EP,), jnp.float32).at[: _E + 1].set(p["fc2_b"])
    qv = jnp.zeros((EP,), jnp.float32).at[: _E + 1].set(p["q_values"])
    gpad = jnp.ones((EP,), jnp.float32).at[: _E + 1].set(p["rnorm_g"])
    bpad = jnp.zeros((EP,), jnp.float32).at[: _E + 1].set(p["rnorm_b"])
    lane_kill = jnp.where(jnp.arange(EP) < _E + 1, 0.0, -jnp.inf)

    visit = jnp.zeros((_E,), jnp.float32)
    for _ in range(_PATH):
        ext = jnp.concatenate([visit >= _MAXV, jnp.zeros((1,), bool)])
        mpad = jnp.zeros((EP,), jnp.float32).at[: _E + 1].set(
            jnp.where(ext, -jnp.inf, 0.0))
        bias = (bpad + mpad + qv + lane_kill).reshape(1, EP)
        act = _router(x, p["fc1_W"], p["fc1_b"], w2p, b2p,
                      gpad.reshape(1, EP), bias)
        action = act[0, 0]
        e = jnp.minimum(action, _E - 1)
        x = lax.cond(action < _E, lambda xx: _expert(p, e, xx),
                     lambda xx: xx, x)
        visit = visit + jax.nn.one_hot(action, _E + 1)[:_E]

    xb = _ln(x, p["fnorm_g"], p["fnorm_b"], out_dtype=jnp.bfloat16)
    logits = _mm(xb, p["lm_W"], p["lm_b"], bn=640)
    return logits.reshape(Bz, S, -1)


# speedup vs baseline: 1.1098x; 1.0019x over previous
import functools
import math

import jax
import jax.numpy as jnp
import numpy as np
from jax import lax
from jax.experimental import pallas as pl
from jax.experimental.pallas import tpu as pltpu
from jax.experimental.pallas import tpu_sc as plsc

_D = 768
_E = 8
_NH = 12
_FF = 3072
_L = 2
_RH = 512
_PATH = 2
_MAXV = 1
_HD = _D // _NH


def _sc_embed(table, ids):
    S = ids.shape[0]
    D = table.shape[1]
    info = plsc.get_sparse_core_info()
    nw = info.num_cores * info.num_subcores
    b_per_w = S // nw
    mesh = plsc.VectorSubcoreMesh(core_axis_name="c", subcore_axis_name="s")

    @functools.partial(
        pl.kernel,
        out_type=jax.ShapeDtypeStruct((S, D), jnp.float32),
        mesh=mesh,
        scratch_types=[
            pltpu.VMEM((b_per_w,), jnp.int32),
            pltpu.VMEM((b_per_w, D), jnp.float32),
            pltpu.SemaphoreType.DMA,
        ],
    )
    def k(table_hbm, idx_hbm, out_hbm, idx_v, rows_v, sem):
        wid = lax.axis_index("s") * info.num_cores + lax.axis_index("c")
        base = wid * b_per_w
        pltpu.sync_copy(idx_hbm.at[pl.ds(base, b_per_w)], idx_v)
        pltpu.async_copy(table_hbm.at[idx_v], rows_v, sem).wait()
        pltpu.sync_copy(rows_v, out_hbm.at[pl.ds(base, b_per_w)])

    return k(table, ids)


def _bf(x):
    return x.astype(jnp.bfloat16) if x.dtype != jnp.bfloat16 else x


_NT = (((1,), (1,)), ((), ()))


def _mm(x, w, b, act=None, res=None, bn=256, out_dtype=jnp.float32):
    M, K = x.shape
    N = w.shape[0]

    def body(x_ref, w_ref, b_ref, *rest):
        if res is not None:
            r_ref, o_ref = rest
        else:
            (o_ref,) = rest
        acc = lax.dot_general(_bf(x_ref[...]), _bf(w_ref[...]), _NT,
                              preferred_element_type=jnp.float32)
        acc = acc + b_ref[...]
        if act == "relu":
            acc = jnp.maximum(acc, 0.0)
        elif act == "sigmoid":
            acc = jax.nn.sigmoid(acc)
        if res is not None:
            acc = acc + r_ref[...]
        o_ref[...] = acc.astype(out_dtype)

    in_specs = [
        pl.BlockSpec((M, K), lambda j: (0, 0)),
        pl.BlockSpec((bn, K), lambda j: (j, 0)),
        pl.BlockSpec((1, bn), lambda j: (0, j)),
    ]
    args = [x, w, b.reshape(1, N)]
    if res is not None:
        in_specs.append(pl.BlockSpec((M, bn), lambda j: (0, j)))
        args.append(res)
    return pl.pallas_call(
        body,
        grid=(N // bn,),
        in_specs=in_specs,
        out_specs=pl.BlockSpec((M, bn), lambda j: (0, j)),
        out_shape=jax.ShapeDtypeStruct((M, N), out_dtype),
    )(*args)


def _emm(x, W, l, e_arr, b, act=None, res=None, gate_split=None, bn=384,
         out_dtype=jnp.float32, dual=False):
    M, _ = x.shape
    N, K = W.shape[2], W.shape[3]

    def body(e_ref, x_ref, w_ref, b_ref, *rest):
        rest = list(rest)
        r_ref = rest.pop(0) if res is not None else None
        o_ref = rest.pop(0)
        o2_ref = rest.pop(0) if dual else None
        wb = _bf(w_ref[0, 0])
        if gate_split is not None:
            acc = lax.dot_general(_bf(x_ref[...]), wb[:, :gate_split], _NT,
                                  preferred_element_type=jnp.float32)
            acc = acc + lax.dot_general(_bf(r_ref[...]), wb[:, gate_split:],
                                        _NT, preferred_element_type=jnp.float32)
        else:
            acc = lax.dot_general(_bf(x_ref[...]), wb, _NT,
                                  preferred_element_type=jnp.float32)
        acc = acc + b_ref[...]
        if act == "relu":
            acc = jnp.maximum(acc, 0.0)
        elif act == "sigmoid":
            acc = jax.nn.sigmoid(acc)
        if res is not None and gate_split is None:
            acc = acc + r_ref[...]
        o_ref[...] = acc.astype(out_dtype)
        if dual:
            o2_ref[...] = acc.astype(jnp.bfloat16)

    in_specs = [
        pl.BlockSpec((M, x.shape[1]), lambda j, e: (0, 0)),
        pl.BlockSpec((1, 1, bn, K), lambda j, e: (e[0], l, j, 0)),
        pl.BlockSpec((1, bn), lambda j, e: (0, j)),
    ]
    args = [x, W, b.reshape(1, N)]
    if res is not None:
        if gate_split is not None:
            in_specs.append(
                pl.BlockSpec((M, K - gate_split), lambda j, e: (0, 0)))
        else:
            in_specs.append(pl.BlockSpec((M, bn), lambda j, e: (0, j)))
        args.append(res)
    out_shape = jax.ShapeDtypeStruct((M, N), out_dtype)
    out_specs = pl.BlockSpec((M, bn), lambda j, e: (0, j))
    if dual:
        out_shape = (out_shape, jax.ShapeDtypeStruct((M, N), jnp.bfloat16))
        out_specs = (out_specs, pl.BlockSpec((M, bn), lambda j, e: (0, j)))
    grid_spec = pltpu.PrefetchScalarGridSpec(
        num_scalar_prefetch=1,
        grid=(N // bn,),
        in_specs=in_specs,
        out_specs=out_specs,
    )
    return pl.pallas_call(
        body,
        grid_spec=grid_spec,
        out_shape=out_shape,
    )(e_arr, *args)


def _emm_ln(x, W, l, e_arr, b, res, lng, lnb, bm=512):
    M, K = x.shape
    N = W.shape[2]

    def body(e_ref, x_ref, w_ref, b_ref, r_ref, g_ref, bb_ref,
             o_ref, h_ref, hb_ref):
        acc = lax.dot_general(_bf(x_ref[...]), _bf(w_ref[0, 0]), _NT,
                              preferred_element_type=jnp.float32)
        xn = acc + b_ref[...] + r_ref[...]
        o_ref[...] = xn
        m = xn.mean(-1, keepdims=True)
        v = ((xn - m) ** 2).mean(-1, keepdims=True)
        h = (xn - m) / jnp.sqrt(v + 1e-5) * g_ref[...] + bb_ref[...]
        h_ref[...] = h
        hb_ref[...] = h.astype(jnp.bfloat16)

    blk = pl.BlockSpec((bm, N), lambda i, e: (i, 0))
    grid_spec = pltpu.PrefetchScalarGridSpec(
        num_scalar_prefetch=1,
        grid=(M // bm,),
        in_specs=[
            pl.BlockSpec((bm, K), lambda i, e: (i, 0)),
            pl.BlockSpec((1, 1, N, K), lambda i, e: (e[0], l, 0, 0)),
            pl.BlockSpec((1, N), lambda i, e: (0, 0)),
            blk,
            pl.BlockSpec((1, N), lambda i, e: (0, 0)),
            pl.BlockSpec((1, N), lambda i, e: (0, 0)),
        ],
        out_specs=(blk, blk, blk),
    )
    return pl.pallas_call(
        body,
        grid_spec=grid_spec,
        out_shape=(jax.ShapeDtypeStruct((M, N), jnp.float32),
                   jax.ShapeDtypeStruct((M, N), jnp.float32),
                   jax.ShapeDtypeStruct((M, N), jnp.bfloat16)),
    )(e_arr, x, W, b.reshape(1, N), res, lng.reshape(1, N), lnb.reshape(1, N))


def _ln(x, g, b, bm=512, dual=False, out_dtype=jnp.float32):
    M, D = x.shape

    def body(x_ref, g_ref, b_ref, *outs):
        xb = x_ref[...]
        m = xb.mean(-1, keepdims=True)
        v = ((xb - m) ** 2).mean(-1, keepdims=True)
        y = (xb - m) / jnp.sqrt(v + 1e-5) * g_ref[...] + b_ref[...]
        if dual:
            outs[0][...] = y
            outs[1][...] = y.astype(jnp.bfloat16)
        else:
            outs[0][...] = y.astype(out_dtype)

    out_shape = jax.ShapeDtypeStruct((M, D), out_dtype)
    out_specs = pl.BlockSpec((bm, D), lambda i: (i, 0))
    if dual:
        out_shape = (jax.ShapeDtypeStruct((M, D), jnp.float32),
                     jax.ShapeDtypeStruct((M, D), jnp.bfloat16))
        out_specs = (out_specs, pl.BlockSpec((bm, D), lambda i: (i, 0)))
    return pl.pallas_call(
        body,
        grid=(M // bm,),
        in_specs=[
            pl.BlockSpec((bm, D), lambda i: (i, 0)),
            pl.BlockSpec((1, D), lambda i: (0, 0)),
            pl.BlockSpec((1, D), lambda i: (0, 0)),
        ],
        out_specs=out_specs,
        out_shape=out_shape,
    )(x, g.reshape(1, D), b.reshape(1, D))


def _add(a, b, bm=512):
    M, D = a.shape

    def body(a_ref, b_ref, o_ref):
        o_ref[...] = a_ref[...] + b_ref[...]

    return pl.pallas_call(
        body,
        grid=(M // bm,),
        in_specs=[pl.BlockSpec((bm, D), lambda i: (i, 0))] * 2,
        out_specs=pl.BlockSpec((bm, D), lambda i: (i, 0)),
        out_shape=jax.ShapeDtypeStruct((M, D), jnp.float32),
    )(a, b)


def _gate_combine(x, h, gt, a, gg, gb, n2g, n2b, bm=512):
    M, D = x.shape

    def body(x_ref, h_ref, gt_ref, a_ref, g_ref, b_ref, g2_ref, b2_ref,
             o_ref, o2_ref):
        u = h_ref[...] + gt_ref[...] * a_ref[...]
        m = u.mean(-1, keepdims=True)
        v = ((u - m) ** 2).mean(-1, keepdims=True)
        xn = x_ref[...] + (u - m) / jnp.sqrt(v + 1e-5) * g_ref[...] + b_ref[...]
        o_ref[...] = xn
        m2 = xn.mean(-1, keepdims=True)
        v2 = ((xn - m2) ** 2).mean(-1, keepdims=True)
        h3 = (xn - m2) / jnp.sqrt(v2 + 1e-5) * g2_ref[...] + b2_ref[...]
        o2_ref[...] = h3.astype(jnp.bfloat16)

    vec = pl.BlockSpec((1, D), lambda i: (0, 0))
    blk = pl.BlockSpec((bm, D), lambda i: (i, 0))
    return pl.pallas_call(
        body,
        grid=(M // bm,),
        in_specs=[blk] * 4 + [vec] * 4,
        out_specs=(blk, blk),
        out_shape=(jax.ShapeDtypeStruct((M, D), jnp.float32),
                   jax.ShapeDtypeStruct((M, D), jnp.bfloat16)),
    )(x, h, gt, a, gg.reshape(1, D), gb.reshape(1, D),
      n2g.reshape(1, D), n2b.reshape(1, D))


def _attn_qkv(qkv):
    S = qkv.shape[0]
    bq = 256
    scale = 1.0 / math.sqrt(_HD)
    hp = _NH // 2

    def body(q_ref, k_ref, v_ref, o_ref):
        q = q_ref[...] * jnp.bfloat16(scale)
        k = k_ref[...]
        v = v_ref[...]

        def one(qh, kh, vh):
            att = lax.dot_general(qh, kh, _NT,
                                  preferred_element_type=jnp.float32)
            p = jnp.exp(att)
            r = 1.0 / p.sum(-1, keepdims=True)
            return jnp.dot(_bf(p), vh, preferred_element_type=jnp.float32) * r

        o1 = one(q[:, :_HD], k[:, :_HD], v[:, :_HD])
        o2 = one(q[:, _HD:], k[:, _HD:], v[:, _HD:])
        o_ref[...] = jnp.concatenate([o1, o2], axis=1).astype(jnp.bfloat16)

    return pl.pallas_call(
        body,
        grid=(hp, S // bq),
        in_specs=[
            pl.BlockSpec((bq, 2 * _HD), lambda h, i: (i, h)),
            pl.BlockSpec((S, 2 * _HD), lambda h, i: (0, hp + h)),
            pl.BlockSpec((S, 2 * _HD), lambda h, i: (0, 2 * hp + h)),
        ],
        out_specs=pl.BlockSpec((bq, 2 * _HD), lambda h, i: (i, h)),
        out_shape=jax.ShapeDtypeStruct((S, _D), jnp.bfloat16),
    )(qkv, qkv, qkv)


def _router(x, w1, b1, w2, b2, gain, bias):
    S, D = x.shape
    RH = w1.shape[0]
    EP = w2.shape[0]

    def body(x_ref, w1_ref, b1_ref, w2_ref, b2_ref, g_ref, bias_ref, o_ref):
        s = jnp.mean(x_ref[...], axis=0, keepdims=True)
        h = lax.dot_general(s, w1_ref[...], _NT,
                            preferred_element_type=jnp.float32) + b1_ref[...]
        h = jnp.maximum(h, 0.0)
        lg = lax.dot_general(h, w2_ref[...], _NT,
                             preferred_element_type=jnp.float32) + b2_ref[...]
        lg = jnp.clip(lg, -10.0, 10.0)
        lane = lax.broadcasted_iota(jnp.int32, (1, EP), 1)
        valid = lane < (_E + 1)
        cnt = float(_E + 1)
        m = jnp.sum(jnp.where(valid, lg, 0.0)) / cnt
        var = jnp.sum(jnp.where(valid, (lg - m) ** 2, 0.0)) / cnt
        lgn = (lg - m) / jnp.sqrt(var + 1e-5)
        score = lgn * g_ref[...] + bias_ref[...]
        top = jnp.max(score, axis=1, keepdims=True)
        cand = jnp.where(score >= top, lane, EP)
        o_ref[...] = jnp.min(cand, axis=1, keepdims=True)

    return pl.pallas_call(
        body,
        grid=(1,),
        in_specs=[
            pl.BlockSpec((S, D), lambda i: (0, 0)),
            pl.BlockSpec((RH, D), lambda i: (0, 0)),
            pl.BlockSpec((1, RH), lambda i: (0, 0)),
            pl.BlockSpec((EP, RH), lambda i: (0, 0)),
            pl.BlockSpec((1, EP), lambda i: (0, 0)),
            pl.BlockSpec((1, EP), lambda i: (0, 0)),
            pl.BlockSpec((1, EP), lambda i: (0, 0)),
        ],
        out_specs=pl.BlockSpec((1, 1), lambda i: (0, 0)),
        out_shape=jax.ShapeDtypeStruct((1, 1), jnp.int32),
    )(x, w1, b1.reshape(1, RH), w2, b2.reshape(1, EP), gain, bias)


def _pe_table(seq, d):
    pos = np.arange(seq)[:, None].astype(np.float32)
    div = np.exp(np.arange(0, d, 2).astype(np.float32) * (-math.log(10000.0) / d))
    pe = np.zeros((seq, d), np.float32)
    pe[:, 0::2] = np.sin(pos * div)
    pe[:, 1::2] = np.cos(pos * div)
    return jnp.asarray(pe)


def _expert(p, e, x):
    e_arr = e.reshape(1).astype(jnp.int32)

    def sl(name):
        return lax.dynamic_index_in_dim(p[name], e, 0, keepdims=False)

    bqkv, bo = sl("attn_bqkv"), sl("attn_bo")
    gb = sl("gate_b")
    gag, gab = sl("ga_g"), sl("ga_b")
    n1g, n1b = sl("norm1_g"), sl("norm1_b")
    n2g, n2b = sl("norm2_g"), sl("norm2_b")
    b1, b2 = sl("ffn_b1"), sl("ffn_b2")
    tag = sl("tag")

    h, hb = _ln(x, n1g[0], n1b[0], dual=True)
    for l in range(_L):
        qkv = _emm(hb, p["attn_Wqkv"], l, e_arr, bqkv[l],
                   out_dtype=jnp.bfloat16)
        ao = _attn_qkv(qkv)
        a, ab = _emm(ao, p["attn_Wo"], l, e_arr, bo[l], dual=True)
        gt = _emm(hb, p["gate_W"], l, e_arr, gb[l], act="sigmoid",
                  res=ab, gate_split=_D)
        x, h3 = _gate_combine(x, h, gt, a, gag[l], gab[l], n2g[l], n2b[l])
        f1 = _emm(h3, p["ffn_W1"], l, e_arr, b1[l], act="relu",
                  out_dtype=jnp.bfloat16)
        if l == _L - 1:
            x = _emm(f1, p["ffn_W2"], l, e_arr, b2[l] + tag, res=x)
        else:
            x, h, hb = _emm_ln(f1, p["ffn_W2"], l, e_arr, b2[l], x,
                               n1g[l + 1], n1b[l + 1])
    return x


def kernel(params, input_ids):
    p = params
    Bz, S = input_ids.shape
    ids = input_ids.reshape(S)

    emb = _sc_embed(p["embedding"], ids)
    x = _add(emb, _pe_table(S, _D))

    EP = 16
    w2p = jnp.zeros((EP, _RH), jnp.float32).at[: _E + 1].set(p["fc2_W"])
    b2p = jnp.zeros((EP,), jnp.float32).at[: _E + 1].set(p["fc2_b"])
    qv = jnp.zeros((EP,), jnp.float32).at[: _E + 1].set(p["q_values"])
    gpad = jnp.ones((EP,), jnp.float32).at[: _E + 1].set(p["rnorm_g"])
    bpad = jnp.zeros((EP,), jnp.float32).at[: _E + 1].set(p["rnorm_b"])
    lane_kill = jnp.where(jnp.arange(EP) < _E + 1, 0.0, -jnp.inf)

    visit = jnp.zeros((_E,), jnp.float32)
    for _ in range(_PATH):
        ext = jnp.concatenate([visit >= _MAXV, jnp.zeros((1,), bool)])
        mpad = jnp.zeros((EP,), jnp.float32).at[: _E + 1].set(
            jnp.where(ext, -jnp.inf, 0.0))
        bias = (bpad + mpad + qv + lane_kill).reshape(1, EP)
        act = _router(x, p["fc1_W"], p["fc1_b"], w2p, b2p,
                      gpad.reshape(1, EP), bias)
        action = act[0, 0]
        e = jnp.minimum(action, _E - 1)
        x = lax.cond(action < _E, lambda xx: _expert(p, e, xx),
                     lambda xx: xx, x)
        visit = visit + jax.nn.one_hot(action, _E + 1)[:_E]

    xb = _ln(x, p["fnorm_g"], p["fnorm_b"], out_dtype=jnp.bfloat16)
    logits = _mm(xb, p["lm_W"], p["lm_b"], bn=640)
    return logits.reshape(Bz, S, -1)
